# baseline (speedup 1.0000x reference)
import functools

import numpy as np

import jax
import jax.numpy as jnp
from jax import lax
from jax.experimental import pallas as pl
from jax.experimental.pallas import tpu as pltpu
from jax.experimental.pallas import tpu_sc as plsc

_PATCH = 16
_NSIDE = 32
_N = _NSIDE * _NSIDE
_B = 4
_C = 3
_DIST = 24.0
_NODE_THRES = 4
_K = 8
_GRID = 8


def _graph_constants():
    ps = _PATCH
    ys = np.arange(_NSIDE, dtype=np.float32) * ps + ps // 2
    xs = np.arange(_NSIDE, dtype=np.float32) * ps + ps // 2
    cy, cx = np.meshgrid(ys, xs, indexing="ij")
    centers = np.stack([cx.reshape(-1), cy.reshape(-1)], axis=1)
    diff = centers[:, None, :] - centers[None, :, :]
    d = np.sqrt(np.sum(diff * diff, axis=-1))
    upper = np.arange(_N)[:, None] < np.arange(_N)[None, :]
    src, dst = np.nonzero((d < _DIST) & upper)
    ei = np.stack([src, dst], axis=0)
    ei1 = np.concatenate([ei, ei[::-1]], axis=1)

    block = ps * 4
    comm = (centers[:, 1] // block).astype(np.int32) * _GRID + (
        centers[:, 0] // block).astype(np.int32)
    same = comm[src] == comm[dst]
    ei_f = np.stack([src[same], dst[same]], axis=0)
    ei2 = np.concatenate([ei_f, ei_f[::-1]], axis=1)

    num_comm = _GRID * _GRID
    counts = np.zeros((num_comm,), np.float32)
    np.add.at(counts, comm, 1.0)
    sum_centers = np.zeros((num_comm, 2), np.float32)
    np.add.at(sum_centers, comm, centers)
    kept = np.nonzero(counts >= _NODE_THRES)[0]
    new_nodes = sum_centers[kept] / counts[kept][:, None]
    m = new_nodes.shape[0]
    nd = new_nodes[:, None, :] - new_nodes[None, :, :]
    dd = np.sqrt(np.sum(nd * nd, axis=-1))
    order = np.argsort(dd, axis=1, kind="stable")[:, 1:_K + 1]
    src3 = np.repeat(np.arange(m), _K)
    dst3 = order.reshape(-1)
    ei3 = np.stack([src3, dst3], axis=0)

    e1 = np.concatenate([ei1 + _N * b for b in range(_B)], axis=1)
    e2 = np.concatenate([ei2 + _N * b for b in range(_B)], axis=1)
    e3 = np.concatenate([ei3 + m * b for b in range(_B)], axis=1)
    return (e1.astype(np.int32), e2.astype(np.int32), e3.astype(np.int32))


_EI1_NP, _EI2_NP, _EI3_NP = _graph_constants()


def _sc_body(img_hbm, out1_hbm, out3_hbm, in_v, out_v, acc_v,
             sem_in0, sem_in1, sem_out0, sem_out1):
    wid = lax.axis_index("s") * 2 + lax.axis_index("c")
    b = wid // _GRID
    gy = wid - _GRID * b
    scale = jnp.float32(1.0 / 16.0)

    def slab_in(i, sem):
        return pltpu.async_copy(img_hbm.at[b, :, pl.ds(8 * gy + 2 * i, 2)],
                                in_v.at[i % 2], sem)

    in_descs = {0: slab_in(0, sem_in0)}
    out_descs = {}
    in_sems = (sem_in0, sem_in1)
    out_sems = (sem_out0, sem_out1)

    for i in range(4):
        if i + 1 < 4:
            in_descs[i + 1] = slab_in(i + 1, in_sems[(i + 1) % 2])
        in_descs[i].wait()
        if i >= 2:
            out_descs[i - 2].wait()

        def gx_body(gx, i=i):
            q = gx // 2
            m = gx - 2 * q
            for r in range(48):
                c, dy = divmod(r, 16)
                t, y8 = divmod(dy, 8)
                lt, l0 = r // 8, 16 * (r % 8)
                v0 = in_v[i % 2, c, t, q, y8, pl.ds(64 * m, 16)]
                v1 = in_v[i % 2, c, t, q, y8, pl.ds(64 * m + 16, 16)]
                v2 = in_v[i % 2, c, t, q, y8, pl.ds(64 * m + 32, 16)]
                v3 = in_v[i % 2, c, t, q, y8, pl.ds(64 * m + 48, 16)]
                out_v[i % 2, q, lt, 4 * m, pl.ds(l0, 16)] = v0
                out_v[i % 2, q, lt, 4 * m + 1, pl.ds(l0, 16)] = v1
                out_v[i % 2, q, lt, 4 * m + 2, pl.ds(l0, 16)] = v2
                out_v[i % 2, q, lt, 4 * m + 3, pl.ds(l0, 16)] = v3
                part = ((v0 + v1) + (v2 + v3)) * scale
                if i == 0:
                    acc_v[lt, gx, pl.ds(l0, 16)] = part
                else:
                    plsc.addupdate(acc_v.at[lt, gx, pl.ds(l0, 16)], part)

        plsc.parallel_loop(0, _GRID, unroll=4)(gx_body)
        out_descs[i] = pltpu.async_copy(
            out_v.at[i % 2],
            out1_hbm.at[pl.ds(128 * b + 16 * gy + 4 * i, 4)],
            out_sems[i % 2])

    out_descs[2].wait()
    out_descs[3].wait()
    pltpu.sync_copy(acc_v, out3_hbm.at[_GRID * b + gy])


@functools.lru_cache(maxsize=1)
def _get_sc_call():
    return functools.partial(
        pl.kernel,
        mesh=plsc.VectorSubcoreMesh(core_axis_name="c", subcore_axis_name="s"),
        compiler_params=pltpu.CompilerParams(use_tc_tiling_on_sc=False),
        out_type=[
            jax.ShapeDtypeStruct((_B * _N // 8, 6, 8, 128), jnp.float32),
            jax.ShapeDtypeStruct((_B * _GRID * _GRID // 8, 6, 8, 128), jnp.float32),
        ],
        scratch_types=[
            pltpu.VMEM((2, _C, 2, 4, 8, 128), jnp.float32),
            pltpu.VMEM((2, 4, 6, 8, 128), jnp.float32),
            pltpu.VMEM((6, 8, 128), jnp.float32),
            pltpu.SemaphoreType.DMA,
            pltpu.SemaphoreType.DMA,
            pltpu.SemaphoreType.DMA,
            pltpu.SemaphoreType.DMA,
        ],
    )(_sc_body)


def kernel(img_batch):
    img6 = jnp.transpose(img_batch.reshape(_B, _C, 64, 8, 4, 128),
                         (0, 1, 2, 4, 3, 5))
    out1_t, out3_t = _get_sc_call()(img6)
    x1 = jnp.transpose(out1_t, (0, 2, 1, 3)).reshape(_B * _N, 768)
    x3 = jnp.transpose(out3_t, (0, 2, 1, 3)).reshape(_B * _GRID * _GRID, 768)
    ei1 = jnp.asarray(_EI1_NP)
    ei2 = jnp.asarray(_EI2_NP)
    ei3 = jnp.asarray(_EI3_NP)
    return (x1, ei1, x1, ei2, x3, ei3)


# speedup vs baseline: 1.2091x; 1.2091x over previous
import functools

import numpy as np

import jax
import jax.numpy as jnp
from jax import lax
from jax.experimental import pallas as pl
from jax.experimental.pallas import tpu as pltpu
from jax.experimental.pallas import tpu_sc as plsc

_PATCH = 16
_NSIDE = 32
_N = _NSIDE * _NSIDE
_B = 4
_C = 3
_DIST = 24.0
_NODE_THRES = 4
_K = 8
_GRID = 8


def _graph_constants():
    ps = _PATCH
    ys = np.arange(_NSIDE, dtype=np.float32) * ps + ps // 2
    xs = np.arange(_NSIDE, dtype=np.float32) * ps + ps // 2
    cy, cx = np.meshgrid(ys, xs, indexing="ij")
    centers = np.stack([cx.reshape(-1), cy.reshape(-1)], axis=1)
    diff = centers[:, None, :] - centers[None, :, :]
    d = np.sqrt(np.sum(diff * diff, axis=-1))
    upper = np.arange(_N)[:, None] < np.arange(_N)[None, :]
    src, dst = np.nonzero((d < _DIST) & upper)
    ei = np.stack([src, dst], axis=0)
    ei1 = np.concatenate([ei, ei[::-1]], axis=1)

    block = ps * 4
    comm = (centers[:, 1] // block).astype(np.int32) * _GRID + (
        centers[:, 0] // block).astype(np.int32)
    same = comm[src] == comm[dst]
    ei_f = np.stack([src[same], dst[same]], axis=0)
    ei2 = np.concatenate([ei_f, ei_f[::-1]], axis=1)

    num_comm = _GRID * _GRID
    counts = np.zeros((num_comm,), np.float32)
    np.add.at(counts, comm, 1.0)
    sum_centers = np.zeros((num_comm, 2), np.float32)
    np.add.at(sum_centers, comm, centers)
    kept = np.nonzero(counts >= _NODE_THRES)[0]
    new_nodes = sum_centers[kept] / counts[kept][:, None]
    m = new_nodes.shape[0]
    nd = new_nodes[:, None, :] - new_nodes[None, :, :]
    dd = np.sqrt(np.sum(nd * nd, axis=-1))
    order = np.argsort(dd, axis=1, kind="stable")[:, 1:_K + 1]
    src3 = np.repeat(np.arange(m), _K)
    dst3 = order.reshape(-1)
    ei3 = np.stack([src3, dst3], axis=0)

    e1 = np.concatenate([ei1 + _N * b for b in range(_B)], axis=1)
    e2 = np.concatenate([ei2 + _N * b for b in range(_B)], axis=1)
    e3 = np.concatenate([ei3 + m * b for b in range(_B)], axis=1)
    return (e1.astype(np.int32), e2.astype(np.int32), e3.astype(np.int32))


_EI1_NP, _EI2_NP, _EI3_NP = _graph_constants()


def _sc_body(img_hbm, out1_hbm, out3_hbm, in_v, out_v, acc_v,
             sem_in0, sem_in1, sem_out0, sem_out1):
    wid = lax.axis_index("s") * 2 + lax.axis_index("c")
    b = wid // _GRID
    gy = wid - _GRID * b
    scale = jnp.float32(1.0 / 16.0)

    def slab_in(i, sem):
        return pltpu.async_copy(img_hbm.at[b, :, pl.ds(8 * gy + 2 * i, 2)],
                                in_v.at[i % 2], sem)

    in_descs = {0: slab_in(0, sem_in0)}
    out_descs = {}
    in_sems = (sem_in0, sem_in1)
    out_sems = (sem_out0, sem_out1)

    for i in range(4):
        if i + 1 < 4:
            in_descs[i + 1] = slab_in(i + 1, in_sems[(i + 1) % 2])
        in_descs[i].wait()
        if i >= 2:
            out_descs[i - 2].wait()

        def gx_body(gx, i=i):
            q = gx // 2
            m = gx - 2 * q
            for r in range(48):
                c, dy = divmod(r, 16)
                t, y8 = divmod(dy, 8)
                lt, l0 = r // 8, 16 * (r % 8)
                v0 = in_v[i % 2, c, t, q, y8, pl.ds(64 * m, 16)]
                v1 = in_v[i % 2, c, t, q, y8, pl.ds(64 * m + 16, 16)]
                v2 = in_v[i % 2, c, t, q, y8, pl.ds(64 * m + 32, 16)]
                v3 = in_v[i % 2, c, t, q, y8, pl.ds(64 * m + 48, 16)]
                out_v[i % 2, q, lt, 4 * m, pl.ds(l0, 16)] = v0
                out_v[i % 2, q, lt, 4 * m + 1, pl.ds(l0, 16)] = v1
                out_v[i % 2, q, lt, 4 * m + 2, pl.ds(l0, 16)] = v2
                out_v[i % 2, q, lt, 4 * m + 3, pl.ds(l0, 16)] = v3
                part = ((v0 + v1) + (v2 + v3)) * scale
                if i == 0:
                    acc_v[lt, gx, pl.ds(l0, 16)] = part
                else:
                    plsc.addupdate(acc_v.at[lt, gx, pl.ds(l0, 16)], part)

        plsc.parallel_loop(0, _GRID, unroll=2)(gx_body)
        out_descs[i] = pltpu.async_copy(
            out_v.at[i % 2],
            out1_hbm.at[pl.ds(128 * b + 16 * gy + 4 * i, 4)],
            out_sems[i % 2])

    out_descs[2].wait()
    out_descs[3].wait()
    pltpu.sync_copy(acc_v, out3_hbm.at[_GRID * b + gy])


@functools.lru_cache(maxsize=1)
def _get_sc_call():
    return functools.partial(
        pl.kernel,
        mesh=plsc.VectorSubcoreMesh(core_axis_name="c", subcore_axis_name="s"),
        compiler_params=pltpu.CompilerParams(use_tc_tiling_on_sc=False),
        out_type=[
            jax.ShapeDtypeStruct((_B * _N // 8, 6, 8, 128), jnp.float32),
            jax.ShapeDtypeStruct((_B * _GRID * _GRID // 8, 6, 8, 128), jnp.float32),
        ],
        scratch_types=[
            pltpu.VMEM((2, _C, 2, 4, 8, 128), jnp.float32),
            pltpu.VMEM((2, 4, 6, 8, 128), jnp.float32),
            pltpu.VMEM((6, 8, 128), jnp.float32),
            pltpu.SemaphoreType.DMA,
            pltpu.SemaphoreType.DMA,
            pltpu.SemaphoreType.DMA,
            pltpu.SemaphoreType.DMA,
        ],
    )(_sc_body)


def kernel(img_batch):
    img6 = jnp.transpose(img_batch.reshape(_B, _C, 64, 8, 4, 128),
                         (0, 1, 2, 4, 3, 5))
    out1_t, out3_t = _get_sc_call()(img6)
    x1 = jnp.transpose(out1_t, (0, 2, 1, 3)).reshape(_B * _N, 768)
    x3 = jnp.transpose(out3_t, (0, 2, 1, 3)).reshape(_B * _GRID * _GRID, 768)
    ei1 = jnp.asarray(_EI1_NP)
    ei2 = jnp.asarray(_EI2_NP)
    ei3 = jnp.asarray(_EI3_NP)
    return (x1, ei1, x1, ei2, x3, ei3)


# speedup vs baseline: 1.2362x; 1.0224x over previous
import functools

import numpy as np

import jax
import jax.numpy as jnp
from jax import lax
from jax.experimental import pallas as pl
from jax.experimental.pallas import tpu as pltpu
from jax.experimental.pallas import tpu_sc as plsc

_PATCH = 16
_NSIDE = 32
_N = _NSIDE * _NSIDE
_B = 4
_C = 3
_DIST = 24.0
_NODE_THRES = 4
_K = 8
_GRID = 8


def _graph_constants():
    ps = _PATCH
    ys = np.arange(_NSIDE, dtype=np.float32) * ps + ps // 2
    xs = np.arange(_NSIDE, dtype=np.float32) * ps + ps // 2
    cy, cx = np.meshgrid(ys, xs, indexing="ij")
    centers = np.stack([cx.reshape(-1), cy.reshape(-1)], axis=1)
    diff = centers[:, None, :] - centers[None, :, :]
    d = np.sqrt(np.sum(diff * diff, axis=-1))
    upper = np.arange(_N)[:, None] < np.arange(_N)[None, :]
    src, dst = np.nonzero((d < _DIST) & upper)
    ei = np.stack([src, dst], axis=0)
    ei1 = np.concatenate([ei, ei[::-1]], axis=1)

    block = ps * 4
    comm = (centers[:, 1] // block).astype(np.int32) * _GRID + (
        centers[:, 0] // block).astype(np.int32)
    same = comm[src] == comm[dst]
    ei_f = np.stack([src[same], dst[same]], axis=0)
    ei2 = np.concatenate([ei_f, ei_f[::-1]], axis=1)

    num_comm = _GRID * _GRID
    counts = np.zeros((num_comm,), np.float32)
    np.add.at(counts, comm, 1.0)
    sum_centers = np.zeros((num_comm, 2), np.float32)
    np.add.at(sum_centers, comm, centers)
    kept = np.nonzero(counts >= _NODE_THRES)[0]
    new_nodes = sum_centers[kept] / counts[kept][:, None]
    m = new_nodes.shape[0]
    nd = new_nodes[:, None, :] - new_nodes[None, :, :]
    dd = np.sqrt(np.sum(nd * nd, axis=-1))
    order = np.argsort(dd, axis=1, kind="stable")[:, 1:_K + 1]
    src3 = np.repeat(np.arange(m), _K)
    dst3 = order.reshape(-1)
    ei3 = np.stack([src3, dst3], axis=0)

    e1 = np.concatenate([ei1 + _N * b for b in range(_B)], axis=1)
    e2 = np.concatenate([ei2 + _N * b for b in range(_B)], axis=1)
    e3 = np.concatenate([ei3 + m * b for b in range(_B)], axis=1)
    return (e1.astype(np.int32), e2.astype(np.int32), e3.astype(np.int32))


_EI1_NP, _EI2_NP, _EI3_NP = _graph_constants()


def _sc_body(img_hbm, out1_hbm, out3_hbm, in_v, out_v, acc_v,
             sem_in0, sem_in1, sem_out0, sem_out1):
    wid = lax.axis_index("s") * 2 + lax.axis_index("c")
    b = wid // _GRID
    gy = wid - _GRID * b

    zv = jnp.zeros((16,), jnp.float32)

    @plsc.parallel_loop(0, _GRID)
    def _(gx):
        for r in range(48):
            acc_v[r // 8, gx, pl.ds(16 * (r % 8), 16)] = zv

    def slab_in(i, sem):
        return pltpu.async_copy(img_hbm.at[b, :, pl.ds(8 * gy + 2 * i, 2)],
                                in_v.at[i % 2], sem)

    in_descs = {0: slab_in(0, sem_in0)}
    out_descs = {}
    in_sems = (sem_in0, sem_in1)
    out_sems = (sem_out0, sem_out1)

    for i in range(4):
        if i + 1 < 4:
            in_descs[i + 1] = slab_in(i + 1, in_sems[(i + 1) % 2])
        in_descs[i].wait()
        if i >= 2:
            out_descs[i - 2].wait()

        def gx_body(gx, i=i):
            q = gx // 2
            m = gx - 2 * q
            for r in range(48):
                c, dy = divmod(r, 16)
                t, y8 = divmod(dy, 8)
                lt, l0 = r // 8, 16 * (r % 8)
                v0 = in_v[i % 2, c, t, q, y8, pl.ds(64 * m, 16)]
                v1 = in_v[i % 2, c, t, q, y8, pl.ds(64 * m + 16, 16)]
                v2 = in_v[i % 2, c, t, q, y8, pl.ds(64 * m + 32, 16)]
                v3 = in_v[i % 2, c, t, q, y8, pl.ds(64 * m + 48, 16)]
                out_v[i % 2, q, lt, 4 * m, pl.ds(l0, 16)] = v0
                out_v[i % 2, q, lt, 4 * m + 1, pl.ds(l0, 16)] = v1
                out_v[i % 2, q, lt, 4 * m + 2, pl.ds(l0, 16)] = v2
                out_v[i % 2, q, lt, 4 * m + 3, pl.ds(l0, 16)] = v3
                plsc.addupdate(acc_v.at[lt, gx, pl.ds(l0, 16)],
                               (v0 + v1) + (v2 + v3))

        plsc.parallel_loop(0, _GRID, unroll=2)(gx_body)
        out_descs[i] = pltpu.async_copy(
            out_v.at[i % 2],
            out1_hbm.at[pl.ds(128 * b + 16 * gy + 4 * i, 4)],
            out_sems[i % 2])

    out_descs[2].wait()
    out_descs[3].wait()

    scale = jnp.float32(1.0 / 16.0)

    @plsc.parallel_loop(0, _GRID)
    def _(gx):
        for r in range(48):
            lt, l0 = r // 8, 16 * (r % 8)
            acc_v[lt, gx, pl.ds(l0, 16)] = acc_v[lt, gx, pl.ds(l0, 16)] * scale

    pltpu.sync_copy(acc_v, out3_hbm.at[_GRID * b + gy])


@functools.lru_cache(maxsize=1)
def _get_sc_call():
    return functools.partial(
        pl.kernel,
        mesh=plsc.VectorSubcoreMesh(core_axis_name="c", subcore_axis_name="s"),
        compiler_params=pltpu.CompilerParams(use_tc_tiling_on_sc=False),
        out_type=[
            jax.ShapeDtypeStruct((_B * _N // 8, 6, 8, 128), jnp.float32),
            jax.ShapeDtypeStruct((_B * _GRID * _GRID // 8, 6, 8, 128), jnp.float32),
        ],
        scratch_types=[
            pltpu.VMEM((2, _C, 2, 4, 8, 128), jnp.float32),
            pltpu.VMEM((2, 4, 6, 8, 128), jnp.float32),
            pltpu.VMEM((6, 8, 128), jnp.float32),
            pltpu.SemaphoreType.DMA,
            pltpu.SemaphoreType.DMA,
            pltpu.SemaphoreType.DMA,
            pltpu.SemaphoreType.DMA,
        ],
    )(_sc_body)


def kernel(img_batch):
    img6 = jnp.transpose(img_batch.reshape(_B, _C, 64, 8, 4, 128),
                         (0, 1, 2, 4, 3, 5))
    out1_t, out3_t = _get_sc_call()(img6)
    x1 = jnp.transpose(out1_t, (0, 2, 1, 3)).reshape(_B * _N, 768)
    x3 = jnp.transpose(out3_t, (0, 2, 1, 3)).reshape(_B * _GRID * _GRID, 768)
    ei1 = jnp.asarray(_EI1_NP)
    ei2 = jnp.asarray(_EI2_NP)
    ei3 = jnp.asarray(_EI3_NP)
    return (x1, ei1, x1, ei2, x3, ei3)
